# speedup vs baseline: 2.4163x; 2.4163x over previous
import functools
import math

import jax
import jax.numpy as jnp
from jax import lax
from jax.experimental import pallas as pl
from jax.experimental.pallas import tpu as pltpu
from jax.experimental.pallas import tpu_sc as plsc

D_MODEL = 128
LANES = 16
NUM_CORES = 2
NUM_SUBCORES = 16
NUM_WORKERS = NUM_CORES * NUM_SUBCORES
CHUNK = 128
SCALE = math.sqrt(D_MODEL)


@functools.partial(jax.jit, static_argnames=("nchunks",))
def _lookup(idx, table, nchunks):
    mesh = plsc.VectorSubcoreMesh(core_axis_name="c", subcore_axis_name="s")
    b_per_w = nchunks * CHUNK
    B = NUM_WORKERS * b_per_w

    @functools.partial(
        pl.kernel,
        mesh=mesh,
        out_type=jax.ShapeDtypeStruct((B, D_MODEL), jnp.float32),
        scratch_types=[
            pltpu.VMEM((nchunks, CHUNK), jnp.int32),
            pltpu.VMEM((CHUNK, D_MODEL), jnp.float32),
            pltpu.SemaphoreType.DMA,
        ],
    )
    def k(idx_hbm, table_hbm, out_hbm, idx_v, rows_v, gsem):
        wid = lax.axis_index("s") * NUM_CORES + lax.axis_index("c")
        base = wid * b_per_w
        pltpu.sync_copy(idx_hbm.at[wid], idx_v)

        def chunk_body(j, carry):
            pltpu.async_copy(table_hbm.at[idx_v.at[j]], rows_v, gsem).wait()

            def row_body(r, c2):
                for c in range(D_MODEL // LANES):
                    sl = pl.ds(c * LANES, LANES)
                    rows_v[r, sl] = rows_v[r, sl] * SCALE
                return c2

            lax.fori_loop(0, CHUNK, row_body, 0, unroll=2)
            pltpu.sync_copy(rows_v, out_hbm.at[pl.ds(base + j * CHUNK, CHUNK)])
            return carry

        lax.fori_loop(0, nchunks, chunk_body, 0)

    return k(idx, table)


def kernel(x, table):
    b0, b1 = x.shape
    n = b0 * b1
    idx = x.reshape(n).astype(jnp.int32)
    group = NUM_WORKERS * CHUNK
    n_pad = -(-n // group) * group
    if n_pad != n:
        idx = jnp.pad(idx, (0, n_pad - n))
    nchunks = n_pad // group
    idx = idx.reshape(NUM_WORKERS, nchunks, CHUNK)
    out = _lookup(idx, table, nchunks)
    return out[:n].reshape(b0, b1, D_MODEL)


# speedup vs baseline: 2.9503x; 1.2210x over previous
import functools
import math

import jax
import jax.numpy as jnp
from jax import lax
from jax.experimental import pallas as pl
from jax.experimental.pallas import tpu as pltpu
from jax.experimental.pallas import tpu_sc as plsc

D_MODEL = 128
LANES = 16
NUM_CORES = 2
NUM_SUBCORES = 16
NUM_WORKERS = NUM_CORES * NUM_SUBCORES
CHUNK = 128
SCALE = math.sqrt(D_MODEL)


@functools.partial(jax.jit, static_argnames=("nchunks",))
def _lookup(idx, table, nchunks):
    mesh = plsc.VectorSubcoreMesh(core_axis_name="c", subcore_axis_name="s")
    b_per_w = nchunks * CHUNK
    B = NUM_WORKERS * b_per_w
    nbuf = next(d for d in (5, 4, 3, 2, 1) if nchunks % d == 0)
    lead = min(2, nbuf - 1)

    @functools.partial(
        pl.kernel,
        mesh=mesh,
        out_type=jax.ShapeDtypeStruct((B, D_MODEL), jnp.float32),
        scratch_types=[
            pltpu.VMEM((nchunks, CHUNK), jnp.int32),
            pltpu.VMEM((nbuf, CHUNK, D_MODEL), jnp.float32),
            pltpu.SemaphoreType.DMA((nbuf,)),
            pltpu.SemaphoreType.DMA((nbuf,)),
        ],
    )
    def k(idx_hbm, table_hbm, out_hbm, idx_v, rows_v, gsem, osem):
        wid = lax.axis_index("s") * NUM_CORES + lax.axis_index("c")
        base = wid * b_per_w
        pltpu.sync_copy(idx_hbm.at[wid], idx_v)

        def start_gather(j, b):
            pltpu.async_copy(table_hbm.at[idx_v.at[j]], rows_v.at[b], gsem.at[b])

        def wait_gather(j, b):
            pltpu.make_async_copy(
                table_hbm.at[idx_v.at[j]], rows_v.at[b], gsem.at[b]
            ).wait()

        def start_scatter(j, b):
            pltpu.async_copy(
                rows_v.at[b], out_hbm.at[pl.ds(base + j * CHUNK, CHUNK)], osem.at[b]
            )

        def wait_scatter(b):
            pltpu.make_async_copy(
                rows_v.at[b], out_hbm.at[pl.ds(base, CHUNK)], osem.at[b]
            ).wait()

        for j in range(lead):
            start_gather(j, j)

        def outer(j0, carry):
            for db in range(nbuf):
                j = j0 + db
                bb = (db + lead) % nbuf

                @pl.when(jnp.logical_and(j + lead < nchunks, j + lead >= nbuf))
                def _():
                    wait_scatter(bb)

                @pl.when(j + lead < nchunks)
                def _():
                    start_gather(j + lead, bb)

                wait_gather(j, db)

                def row_body(r, c2):
                    for c in range(D_MODEL // LANES):
                        sl = pl.ds(c * LANES, LANES)
                        rows_v[db, r, sl] = rows_v[db, r, sl] * SCALE
                    return c2

                lax.fori_loop(0, CHUNK, row_body, 0, unroll=2)
                start_scatter(j, db)
            return carry

        lax.fori_loop(0, nchunks // nbuf, lambda i, c: outer(i * nbuf, c), 0)

        for b in range(nbuf):
            wait_scatter(b)

    return k(idx, table)


def kernel(x, table):
    b0, b1 = x.shape
    n = b0 * b1
    idx = x.reshape(n).astype(jnp.int32)
    group = NUM_WORKERS * CHUNK
    n_pad = -(-n // group) * group
    if n_pad != n:
        idx = jnp.pad(idx, (0, n_pad - n))
    nchunks = n_pad // group
    idx = idx.reshape(NUM_WORKERS, nchunks, CHUNK)
    out = _lookup(idx, table, nchunks)
    return out[:n].reshape(b0, b1, D_MODEL)


# speedup vs baseline: 2.9572x; 1.0024x over previous
import functools
import math

import jax
import jax.numpy as jnp
from jax import lax
from jax.experimental import pallas as pl
from jax.experimental.pallas import tpu as pltpu
from jax.experimental.pallas import tpu_sc as plsc

D_MODEL = 128
LANES = 16
NUM_CORES = 2
NUM_SUBCORES = 16
NUM_WORKERS = NUM_CORES * NUM_SUBCORES
CHUNK = 128
SCALE = math.sqrt(D_MODEL)


@functools.partial(jax.jit, static_argnames=("nchunks",))
def _lookup(idx, table, nchunks):
    mesh = plsc.VectorSubcoreMesh(core_axis_name="c", subcore_axis_name="s")
    b_per_w = nchunks * CHUNK
    B = NUM_WORKERS * b_per_w
    nbuf = next(d for d in (5, 4, 3, 2, 1) if nchunks % d == 0)
    lead = min(4, nbuf - 1)

    @functools.partial(
        pl.kernel,
        mesh=mesh,
        out_type=jax.ShapeDtypeStruct((B, D_MODEL), jnp.float32),
        scratch_types=[
            pltpu.VMEM((nchunks, CHUNK), jnp.int32),
            pltpu.VMEM((nbuf, CHUNK, D_MODEL), jnp.float32),
            pltpu.SemaphoreType.DMA((nbuf,)),
            pltpu.SemaphoreType.DMA((nbuf,)),
        ],
    )
    def k(idx_hbm, table_hbm, out_hbm, idx_v, rows_v, gsem, osem):
        wid = lax.axis_index("s") * NUM_CORES + lax.axis_index("c")
        base = wid * b_per_w
        pltpu.sync_copy(idx_hbm.at[wid], idx_v)

        def start_gather(j, b):
            pltpu.async_copy(table_hbm.at[idx_v.at[j]], rows_v.at[b], gsem.at[b])

        def wait_gather(j, b):
            pltpu.make_async_copy(
                table_hbm.at[idx_v.at[j]], rows_v.at[b], gsem.at[b]
            ).wait()

        def start_scatter(j, b):
            pltpu.async_copy(
                rows_v.at[b], out_hbm.at[pl.ds(base + j * CHUNK, CHUNK)], osem.at[b]
            )

        def wait_scatter(b):
            pltpu.make_async_copy(
                rows_v.at[b], out_hbm.at[pl.ds(base, CHUNK)], osem.at[b]
            ).wait()

        for j in range(lead):
            start_gather(j, j)

        def outer(j0, carry):
            for db in range(nbuf):
                j = j0 + db
                bb = (db + lead) % nbuf

                @pl.when(jnp.logical_and(j + lead < nchunks, j + lead >= nbuf))
                def _():
                    wait_scatter(bb)

                @pl.when(j + lead < nchunks)
                def _():
                    start_gather(j + lead, bb)

                wait_gather(j, db)

                def row_body(r, c2):
                    for c in range(D_MODEL // LANES):
                        sl = pl.ds(c * LANES, LANES)
                        rows_v[db, r, sl] = rows_v[db, r, sl] * SCALE
                    return c2

                lax.fori_loop(0, CHUNK, row_body, 0, unroll=2)
                start_scatter(j, db)
            return carry

        lax.fori_loop(0, nchunks // nbuf, lambda i, c: outer(i * nbuf, c), 0)

        for b in range(nbuf):
            wait_scatter(b)

    return k(idx, table)


def kernel(x, table):
    b0, b1 = x.shape
    n = b0 * b1
    idx = x.reshape(n).astype(jnp.int32)
    group = NUM_WORKERS * CHUNK
    n_pad = -(-n // group) * group
    if n_pad != n:
        idx = jnp.pad(idx, (0, n_pad - n))
    nchunks = n_pad // group
    idx = idx.reshape(NUM_WORKERS, nchunks, CHUNK)
    out = _lookup(idx, table, nchunks)
    return out[:n].reshape(b0, b1, D_MODEL)


# speedup vs baseline: 2.9599x; 1.0009x over previous
import functools
import math

import jax
import jax.numpy as jnp
from jax import lax
from jax.experimental import pallas as pl
from jax.experimental.pallas import tpu as pltpu
from jax.experimental.pallas import tpu_sc as plsc

D_MODEL = 128
LANES = 16
NUM_CORES = 2
NUM_SUBCORES = 16
NUM_WORKERS = NUM_CORES * NUM_SUBCORES
CHUNK = 128
SCALE = math.sqrt(D_MODEL)


@functools.partial(jax.jit, static_argnames=("nchunks",))
def _lookup(idx, table, nchunks):
    mesh = plsc.VectorSubcoreMesh(core_axis_name="c", subcore_axis_name="s")
    b_per_w = nchunks * CHUNK
    B = NUM_WORKERS * b_per_w
    nbuf = next(d for d in (5, 4, 3, 2, 1) if nchunks % d == 0)
    lead = min(4, nbuf - 1)

    @functools.partial(
        pl.kernel,
        mesh=mesh,
        out_type=jax.ShapeDtypeStruct((B, D_MODEL), jnp.float32),
        scratch_types=[
            pltpu.VMEM((nchunks, CHUNK), jnp.int32),
            pltpu.VMEM((nbuf, CHUNK, D_MODEL), jnp.float32),
            pltpu.SemaphoreType.DMA((nbuf,)),
            pltpu.SemaphoreType.DMA((nbuf,)),
        ],
    )
    def k(idx_hbm, table_hbm, out_hbm, idx_v, rows_v, gsem, osem):
        cid = lax.axis_index("c")
        sid = lax.axis_index("s")
        wid = sid * NUM_CORES + cid
        base = wid * b_per_w
        pltpu.sync_copy(idx_hbm.at[wid], idx_v)

        def start_gather(j, b):
            pltpu.async_copy(table_hbm.at[idx_v.at[j]], rows_v.at[b], gsem.at[b])

        def wait_gather(j, b):
            pltpu.make_async_copy(
                table_hbm.at[idx_v.at[j]], rows_v.at[b], gsem.at[b]
            ).wait()

        def start_scatter(j, b):
            pltpu.async_copy(
                rows_v.at[b], out_hbm.at[pl.ds(base + j * CHUNK, CHUNK)], osem.at[b]
            )

        def wait_scatter(b):
            pltpu.make_async_copy(
                rows_v.at[b], out_hbm.at[pl.ds(base, CHUNK)], osem.at[b]
            ).wait()

        for j in range(lead):
            start_gather(j, j)

        def outer(j0, carry):
            for db in range(nbuf):
                j = j0 + db
                bb = (db + lead) % nbuf

                @pl.when(jnp.logical_and(j + lead < nchunks, j + lead >= nbuf))
                def _():
                    wait_scatter(bb)

                @pl.when(j + lead < nchunks)
                def _():
                    start_gather(j + lead, bb)

                wait_gather(j, db)

                def row_body(r, c2):
                    for c in range(D_MODEL // LANES):
                        sl = pl.ds(c * LANES, LANES)
                        rows_v[db, r, sl] = rows_v[db, r, sl] * SCALE
                    return c2

                lax.fori_loop(0, CHUNK, row_body, 0, unroll=2)
                start_scatter(j, db)
            return carry

        lax.fori_loop(0, nchunks // nbuf, lambda i, c: outer(i * nbuf, c), 0)

        for b in range(nbuf):
            wait_scatter(b)

    return k(idx, table)


def kernel(x, table):
    b0, b1 = x.shape
    n = b0 * b1
    idx = x.reshape(n).astype(jnp.int32)
    group = NUM_WORKERS * CHUNK
    n_pad = -(-n // group) * group
    n_pad = max(n_pad, group)
    if n_pad != n:
        idx = jnp.pad(idx, (0, n_pad - n))
    nchunks = n_pad // group
    idx = idx.reshape(NUM_WORKERS, nchunks, CHUNK)
    out = _lookup(idx, table, nchunks)
    return out[:n].reshape(b0, b1, D_MODEL)
